# Initial kernel scaffold; baseline (speedup 1.0000x reference)
import jax
import jax.numpy as jnp
from jax.experimental import pallas as pl


def kernel(x, edge_index, edge_vals, W1, b1, W2, b2, Wh, bh):
    raise NotImplementedError("write your pallas kernel here")


# speedup vs baseline: 2.8614x; 2.8614x over previous
import functools

import jax
import jax.numpy as jnp
from jax import lax
from jax.experimental import pallas as pl
from jax.experimental.pallas import tpu as pltpu
from jax.experimental.pallas import tpu_sc as plsc

N_NODES = 10000
F = 128
FH = F // 2
N_EDGES = 320000
NC = 2
NS = 16
EPW = N_EDGES // NS
K = 80
NCH = EPW // K
N_PAD = 10240
STRIPE = N_PAD // NS


def _spmm_body(mat0_hbm, mat1_hbm, src_hbm, dst_hbm, vals_hbm, zero_hbm,
               out_hbm, src_v, dst_v, vals_v, rows_v, acc_sh, sem):
    c = lax.axis_index("c")
    s = lax.axis_index("s")

    pltpu.sync_copy(src_hbm.at[s], src_v)
    pltpu.sync_copy(dst_hbm.at[s], dst_v)
    pltpu.sync_copy(vals_hbm.at[s], vals_v)
    pltpu.sync_copy(zero_hbm.at[pl.ds(s * STRIPE, STRIPE)],
                    acc_sh.at[pl.ds(s * STRIPE, STRIPE)])
    plsc.subcore_barrier()

    def chunk_body(j, carry):
        @pl.when(c == 0)
        def _():
            pltpu.async_copy(mat0_hbm.at[src_v.at[j]], rows_v, sem).wait()

        @pl.when(c == 1)
        def _():
            pltpu.async_copy(mat1_hbm.at[src_v.at[j]], rows_v, sem).wait()

        def grp_body(g, carry2):
            vv = vals_v[j, pl.ds(g * 16, 16)]
            for l in range(16):
                v = vv[l]
                e = g * 16 + l
                for b in range(FH // 16):
                    sl = pl.ds(b * 16, 16)
                    rows_v[e, sl] = rows_v[e, sl] * v
            return carry2

        lax.fori_loop(0, K // 16, grp_body, 0)

        pltpu.sync_copy(rows_v, acc_sh.at[dst_v.at[j]], add=True)
        return carry

    lax.fori_loop(0, NCH, chunk_body, 0)
    plsc.subcore_barrier()

    pltpu.sync_copy(acc_sh.at[pl.ds(s * STRIPE, STRIPE)],
                    out_hbm.at[c, pl.ds(s * STRIPE, STRIPE)])


_sc_spmm = functools.partial(
    pl.kernel,
    out_type=jax.ShapeDtypeStruct((NC, N_PAD, FH), jnp.float32),
    mesh=plsc.VectorSubcoreMesh(core_axis_name="c", subcore_axis_name="s"),
    compiler_params=pltpu.CompilerParams(use_tc_tiling_on_sc=False),
    scratch_types=[
        pltpu.VMEM((NCH, K), jnp.int32),
        pltpu.VMEM((NCH, K), jnp.int32),
        pltpu.VMEM((NCH, K), jnp.float32),
        pltpu.VMEM((K, FH), jnp.float32),
        pltpu.VMEM_SHARED((N_PAD, FH), jnp.float32),
        pltpu.SemaphoreType.DMA,
    ],
)(_spmm_body)


def _mm_relu_body(p_ref, w_ref, b_ref, o_ref):
    y = jnp.concatenate([p_ref[0], p_ref[1]], axis=1)
    z = jnp.dot(y, w_ref[...], preferred_element_type=jnp.float32)
    r = jnp.maximum(z + b_ref[...], 0.0)
    o_ref[0] = r[:, :FH]
    o_ref[1] = r[:, FH:]


def _tc_mm_relu(p, w, b):
    rb = 2000
    grid = N_NODES // rb
    return pl.pallas_call(
        _mm_relu_body,
        grid=(grid,),
        in_specs=[
            pl.BlockSpec((NC, rb, FH), lambda i: (0, i, 0)),
            pl.BlockSpec((F, F), lambda i: (0, 0)),
            pl.BlockSpec((1, F), lambda i: (0, 0)),
        ],
        out_specs=pl.BlockSpec((NC, rb, FH), lambda i: (0, i, 0)),
        out_shape=jax.ShapeDtypeStruct((NC, N_NODES, FH), jnp.float32),
    )(p, w, b.reshape(1, F))


def _final_body(p_ref, w2_ref, b2_ref, wh_ref, bh_ref, s_ref, g_ref, acc_ref):
    i = pl.program_id(0)
    y = jnp.concatenate([p_ref[0], p_ref[1]], axis=1)
    z = jnp.dot(y, w2_ref[...], preferred_element_type=jnp.float32)
    h2 = jnp.maximum(z + b2_ref[...], 0.0)
    psum = jnp.sum(h2, axis=0, keepdims=True)

    @pl.when(i == 0)
    def _():
        acc_ref[...] = psum

    @pl.when(i > 0)
    def _():
        acc_ref[...] = acc_ref[...] + psum

    @pl.when(i == pl.num_programs(0) - 1)
    def _():
        g = acc_ref[...] * (1.0 / N_NODES)
        g_ref[...] = g
        s_ref[...] = (
            jnp.dot(g, wh_ref[...], preferred_element_type=jnp.float32)
            + bh_ref[...]
        )


def _tc_final(p, w2, b2, wh, bh):
    rb = 2000
    grid = N_NODES // rb
    nclass = wh.shape[1]
    return pl.pallas_call(
        _final_body,
        grid=(grid,),
        in_specs=[
            pl.BlockSpec((NC, rb, FH), lambda i: (0, i, 0)),
            pl.BlockSpec((F, F), lambda i: (0, 0)),
            pl.BlockSpec((1, F), lambda i: (0, 0)),
            pl.BlockSpec((F, nclass), lambda i: (0, 0)),
            pl.BlockSpec((1, nclass), lambda i: (0, 0)),
        ],
        out_specs=[
            pl.BlockSpec((1, nclass), lambda i: (0, 0)),
            pl.BlockSpec((1, F), lambda i: (0, 0)),
        ],
        out_shape=[
            jax.ShapeDtypeStruct((1, nclass), jnp.float32),
            jax.ShapeDtypeStruct((1, F), jnp.float32),
        ],
        scratch_shapes=[pltpu.VMEM((1, F), jnp.float32)],
    )(p, w2, b2.reshape(1, F), wh, bh.reshape(1, nclass))


def kernel(x, edge_index, edge_vals, W1, b1, W2, b2, Wh, bh):
    dst = edge_index[0].astype(jnp.int32).reshape(NS, NCH, K)
    src = edge_index[1].astype(jnp.int32).reshape(NS, NCH, K)
    ev = edge_vals.reshape(NS, NCH, K)
    zeros = jnp.zeros((N_PAD, FH), jnp.float32)
    p1 = _sc_spmm(x[:, :FH], x[:, FH:], src, dst, ev, zeros)
    h = _tc_mm_relu(p1, W1, b1)
    p2 = _sc_spmm(h[0], h[1], src, dst, ev, zeros)
    scores, g = _tc_final(p2, W2, b2, Wh, bh)
    return (scores, g)


# speedup vs baseline: 3.9520x; 1.3811x over previous
import functools

import jax
import jax.numpy as jnp
from jax import lax
from jax.experimental import pallas as pl
from jax.experimental.pallas import tpu as pltpu
from jax.experimental.pallas import tpu_sc as plsc

N_NODES = 10000
F = 128
FH = F // 2
N_EDGES = 320000
NC = 2
NS = 16
EPW = N_EDGES // NS
K = 80
NCH = EPW // K
N_PAD = 10240
STRIPE = N_PAD // NS


NB = 5
NG = NCH // NB


def _spmm_body(mat0_hbm, mat1_hbm, src_hbm, dst_hbm, vals_hbm, zero_hbm,
               out_hbm, src_v, dst_v, vals_v,
               r0, r1, r2, r3, r4, acc_sh,
               g0, g1, g2, g3, g4, s0, s1, s2, s3, s4):
    c = lax.axis_index("c")
    s = lax.axis_index("s")
    rows = [r0, r1, r2, r3, r4]
    gsem = [g0, g1, g2, g3, g4]
    ssem = [s0, s1, s2, s3, s4]

    pltpu.sync_copy(src_hbm.at[s], src_v)
    pltpu.sync_copy(dst_hbm.at[s], dst_v)
    pltpu.sync_copy(vals_hbm.at[s], vals_v)
    pltpu.sync_copy(zero_hbm.at[pl.ds(s * STRIPE, STRIPE)],
                    acc_sh.at[pl.ds(s * STRIPE, STRIPE)])
    plsc.subcore_barrier()

    def gather(j, b):
        @pl.when(c == 0)
        def _():
            pltpu.async_copy(mat0_hbm.at[src_v.at[j]], rows[b], gsem[b])

        @pl.when(c == 1)
        def _():
            pltpu.async_copy(mat1_hbm.at[src_v.at[j]], rows[b], gsem[b])

    def wait_gather(b):
        pltpu.make_async_copy(mat0_hbm.at[src_v.at[0]], rows[b],
                              gsem[b]).wait()

    def scatter(j, b):
        pltpu.async_copy(rows[b], acc_sh.at[dst_v.at[j]], ssem[b],
                         add=True)

    def wait_scatter(b):
        pltpu.make_async_copy(rows[b], acc_sh.at[dst_v.at[0]],
                              ssem[b]).wait()

    def scale(j, b):
        def grp_body(g, carry2):
            vv = vals_v[j, pl.ds(g * 16, 16)]
            for l in range(16):
                v = vv[l]
                e = g * 16 + l
                for blk in range(FH // 16):
                    sl = pl.ds(blk * 16, 16)
                    rows[b][e, sl] = rows[b][e, sl] * v
            return carry2

        lax.fori_loop(0, K // 16, grp_body, 0)

    def group_body(gi, carry):
        base = gi * NB

        @pl.when(gi > 0)
        def _():
            for b in range(NB):
                wait_scatter(b)

        for b in range(NB):
            gather(base + b, b)
        for b in range(NB):
            wait_gather(b)
            scale(base + b, b)
            scatter(base + b, b)
        return carry

    lax.fori_loop(0, NG, group_body, 0)
    for b in range(NB):
        wait_scatter(b)
    plsc.subcore_barrier()

    pltpu.sync_copy(acc_sh.at[pl.ds(s * STRIPE, STRIPE)],
                    out_hbm.at[c, pl.ds(s * STRIPE, STRIPE)])


_sc_spmm = functools.partial(
    pl.kernel,
    out_type=jax.ShapeDtypeStruct((NC, N_PAD, FH), jnp.float32),
    mesh=plsc.VectorSubcoreMesh(core_axis_name="c", subcore_axis_name="s"),
    compiler_params=pltpu.CompilerParams(use_tc_tiling_on_sc=False),
    scratch_types=(
        [
            pltpu.VMEM((NCH, K), jnp.int32),
            pltpu.VMEM((NCH, K), jnp.int32),
            pltpu.VMEM((NCH, K), jnp.float32),
        ]
        + [pltpu.VMEM((K, FH), jnp.float32) for _ in range(NB)]
        + [pltpu.VMEM_SHARED((N_PAD, FH), jnp.float32)]
        + [pltpu.SemaphoreType.DMA for _ in range(2 * NB)]
    ),
)(_spmm_body)


def _mm_relu_body(p_ref, w_ref, b_ref, o_ref):
    y = jnp.concatenate([p_ref[0], p_ref[1]], axis=1)
    z = jnp.dot(y, w_ref[...], preferred_element_type=jnp.float32)
    r = jnp.maximum(z + b_ref[...], 0.0)
    o_ref[0] = r[:, :FH]
    o_ref[1] = r[:, FH:]


def _tc_mm_relu(p, w, b):
    rb = 2000
    grid = N_NODES // rb
    return pl.pallas_call(
        _mm_relu_body,
        grid=(grid,),
        in_specs=[
            pl.BlockSpec((NC, rb, FH), lambda i: (0, i, 0)),
            pl.BlockSpec((F, F), lambda i: (0, 0)),
            pl.BlockSpec((1, F), lambda i: (0, 0)),
        ],
        out_specs=pl.BlockSpec((NC, rb, FH), lambda i: (0, i, 0)),
        out_shape=jax.ShapeDtypeStruct((NC, N_NODES, FH), jnp.float32),
    )(p, w, b.reshape(1, F))


def _final_body(p_ref, w2_ref, b2_ref, wh_ref, bh_ref, s_ref, g_ref, acc_ref):
    i = pl.program_id(0)
    y = jnp.concatenate([p_ref[0], p_ref[1]], axis=1)
    z = jnp.dot(y, w2_ref[...], preferred_element_type=jnp.float32)
    h2 = jnp.maximum(z + b2_ref[...], 0.0)
    psum = jnp.sum(h2, axis=0, keepdims=True)

    @pl.when(i == 0)
    def _():
        acc_ref[...] = psum

    @pl.when(i > 0)
    def _():
        acc_ref[...] = acc_ref[...] + psum

    @pl.when(i == pl.num_programs(0) - 1)
    def _():
        g = acc_ref[...] * (1.0 / N_NODES)
        g_ref[...] = g
        s_ref[...] = (
            jnp.dot(g, wh_ref[...], preferred_element_type=jnp.float32)
            + bh_ref[...]
        )


def _tc_final(p, w2, b2, wh, bh):
    rb = 2000
    grid = N_NODES // rb
    nclass = wh.shape[1]
    return pl.pallas_call(
        _final_body,
        grid=(grid,),
        in_specs=[
            pl.BlockSpec((NC, rb, FH), lambda i: (0, i, 0)),
            pl.BlockSpec((F, F), lambda i: (0, 0)),
            pl.BlockSpec((1, F), lambda i: (0, 0)),
            pl.BlockSpec((F, nclass), lambda i: (0, 0)),
            pl.BlockSpec((1, nclass), lambda i: (0, 0)),
        ],
        out_specs=[
            pl.BlockSpec((1, nclass), lambda i: (0, 0)),
            pl.BlockSpec((1, F), lambda i: (0, 0)),
        ],
        out_shape=[
            jax.ShapeDtypeStruct((1, nclass), jnp.float32),
            jax.ShapeDtypeStruct((1, F), jnp.float32),
        ],
        scratch_shapes=[pltpu.VMEM((1, F), jnp.float32)],
    )(p, w2, b2.reshape(1, F), wh, bh.reshape(1, nclass))


def kernel(x, edge_index, edge_vals, W1, b1, W2, b2, Wh, bh):
    dst = edge_index[0].astype(jnp.int32).reshape(NS, NCH, K)
    src = edge_index[1].astype(jnp.int32).reshape(NS, NCH, K)
    ev = edge_vals.reshape(NS, NCH, K)
    zeros = jnp.zeros((N_PAD, FH), jnp.float32)
    p1 = _sc_spmm(x[:, :FH], x[:, FH:], src, dst, ev, zeros)
    h = _tc_mm_relu(p1, W1, b1)
    p2 = _sc_spmm(h[0], h[1], src, dst, ev, zeros)
    scores, g = _tc_final(p2, W2, b2, Wh, bh)
    return (scores, g)


# speedup vs baseline: 7.0094x; 1.7736x over previous
import functools

import jax
import jax.numpy as jnp
from jax import lax
from jax.experimental import pallas as pl
from jax.experimental.pallas import tpu as pltpu
from jax.experimental.pallas import tpu_sc as plsc

N_NODES = 10000
F = 128
FH = F // 2
N_EDGES = 320000
NC = 2
NS = 16
EPW = N_EDGES // NS
K = 80
NCH = EPW // K
N_PAD = 10240
STRIPE = N_PAD // NS


NB = 5
NG = NCH // NB


def _spmm_body(mat0_hbm, mat1_hbm, src_hbm, dst_hbm, vals_hbm, zero_hbm,
               out_hbm, src_v, dst_v, vals_v,
               r0, r1, r2, r3, r4, acc_sh,
               g0, g1, g2, g3, g4, s0, s1, s2, s3, s4):
    c = lax.axis_index("c")
    s = lax.axis_index("s")
    rows = [r0, r1, r2, r3, r4]
    gsem = [g0, g1, g2, g3, g4]
    ssem = [s0, s1, s2, s3, s4]

    pltpu.sync_copy(src_hbm.at[s], src_v)
    pltpu.sync_copy(dst_hbm.at[s], dst_v)
    pltpu.sync_copy(vals_hbm.at[s], vals_v)
    pltpu.sync_copy(zero_hbm.at[pl.ds(s * STRIPE, STRIPE)],
                    acc_sh.at[pl.ds(s * STRIPE, STRIPE)])
    plsc.subcore_barrier()

    def gather(j, b):
        @pl.when(c == 0)
        def _():
            pltpu.async_copy(mat0_hbm.at[src_v.at[j]], rows[b], gsem[b])

        @pl.when(c == 1)
        def _():
            pltpu.async_copy(mat1_hbm.at[src_v.at[j]], rows[b], gsem[b])

    def wait_gather(b):
        pltpu.make_async_copy(mat0_hbm.at[src_v.at[0]], rows[b],
                              gsem[b]).wait()

    def scatter(j, b):
        pltpu.async_copy(rows[b], acc_sh.at[dst_v.at[j]], ssem[b],
                         add=True)

    def wait_scatter(b):
        pltpu.make_async_copy(rows[b], acc_sh.at[dst_v.at[0]],
                              ssem[b]).wait()

    def scale(j, b):
        @plsc.parallel_loop(0, K // 16, unroll=K // 16)
        def grp_body(g):
            vv = vals_v[j, pl.ds(g * 16, 16)]
            for l in range(16):
                v = vv[l]
                e = g * 16 + l
                for blk in range(FH // 16):
                    sl = pl.ds(blk * 16, 16)
                    rows[b][e, sl] = rows[b][e, sl] * v

    def group_body(gi, carry):
        base = gi * NB

        @pl.when(gi > 0)
        def _():
            for b in range(NB):
                wait_scatter(b)

        for b in range(NB):
            gather(base + b, b)
        for b in range(NB):
            wait_gather(b)
            scale(base + b, b)
            scatter(base + b, b)
        return carry

    lax.fori_loop(0, NG, group_body, 0)
    for b in range(NB):
        wait_scatter(b)
    plsc.subcore_barrier()

    pltpu.sync_copy(acc_sh.at[pl.ds(s * STRIPE, STRIPE)],
                    out_hbm.at[c, pl.ds(s * STRIPE, STRIPE)])


_sc_spmm = functools.partial(
    pl.kernel,
    out_type=jax.ShapeDtypeStruct((NC, N_PAD, FH), jnp.float32),
    mesh=plsc.VectorSubcoreMesh(core_axis_name="c", subcore_axis_name="s"),
    compiler_params=pltpu.CompilerParams(use_tc_tiling_on_sc=False),
    scratch_types=(
        [
            pltpu.VMEM((NCH, K), jnp.int32),
            pltpu.VMEM((NCH, K), jnp.int32),
            pltpu.VMEM((NCH, K), jnp.float32),
        ]
        + [pltpu.VMEM((K, FH), jnp.float32) for _ in range(NB)]
        + [pltpu.VMEM_SHARED((N_PAD, FH), jnp.float32)]
        + [pltpu.SemaphoreType.DMA for _ in range(2 * NB)]
    ),
)(_spmm_body)


def _mm_relu_body(p_ref, w_ref, b_ref, o_ref):
    y = jnp.concatenate([p_ref[0], p_ref[1]], axis=1)
    z = jnp.dot(y, w_ref[...], preferred_element_type=jnp.float32)
    r = jnp.maximum(z + b_ref[...], 0.0)
    o_ref[0] = r[:, :FH]
    o_ref[1] = r[:, FH:]


def _tc_mm_relu(p, w, b):
    rb = 2000
    grid = N_NODES // rb
    return pl.pallas_call(
        _mm_relu_body,
        grid=(grid,),
        in_specs=[
            pl.BlockSpec((NC, rb, FH), lambda i: (0, i, 0)),
            pl.BlockSpec((F, F), lambda i: (0, 0)),
            pl.BlockSpec((1, F), lambda i: (0, 0)),
        ],
        out_specs=pl.BlockSpec((NC, rb, FH), lambda i: (0, i, 0)),
        out_shape=jax.ShapeDtypeStruct((NC, N_NODES, FH), jnp.float32),
    )(p, w, b.reshape(1, F))


def _final_body(p_ref, w2_ref, b2_ref, wh_ref, bh_ref, s_ref, g_ref, acc_ref):
    i = pl.program_id(0)
    y = jnp.concatenate([p_ref[0], p_ref[1]], axis=1)
    z = jnp.dot(y, w2_ref[...], preferred_element_type=jnp.float32)
    h2 = jnp.maximum(z + b2_ref[...], 0.0)
    psum = jnp.sum(h2, axis=0, keepdims=True)

    @pl.when(i == 0)
    def _():
        acc_ref[...] = psum

    @pl.when(i > 0)
    def _():
        acc_ref[...] = acc_ref[...] + psum

    @pl.when(i == pl.num_programs(0) - 1)
    def _():
        g = acc_ref[...] * (1.0 / N_NODES)
        g_ref[...] = g
        s_ref[...] = (
            jnp.dot(g, wh_ref[...], preferred_element_type=jnp.float32)
            + bh_ref[...]
        )


def _tc_final(p, w2, b2, wh, bh):
    rb = 2000
    grid = N_NODES // rb
    nclass = wh.shape[1]
    return pl.pallas_call(
        _final_body,
        grid=(grid,),
        in_specs=[
            pl.BlockSpec((NC, rb, FH), lambda i: (0, i, 0)),
            pl.BlockSpec((F, F), lambda i: (0, 0)),
            pl.BlockSpec((1, F), lambda i: (0, 0)),
            pl.BlockSpec((F, nclass), lambda i: (0, 0)),
            pl.BlockSpec((1, nclass), lambda i: (0, 0)),
        ],
        out_specs=[
            pl.BlockSpec((1, nclass), lambda i: (0, 0)),
            pl.BlockSpec((1, F), lambda i: (0, 0)),
        ],
        out_shape=[
            jax.ShapeDtypeStruct((1, nclass), jnp.float32),
            jax.ShapeDtypeStruct((1, F), jnp.float32),
        ],
        scratch_shapes=[pltpu.VMEM((1, F), jnp.float32)],
    )(p, w2, b2.reshape(1, F), wh, bh.reshape(1, nclass))


def kernel(x, edge_index, edge_vals, W1, b1, W2, b2, Wh, bh):
    dst = edge_index[0].astype(jnp.int32).reshape(NS, NCH, K)
    src = edge_index[1].astype(jnp.int32).reshape(NS, NCH, K)
    ev = edge_vals.reshape(NS, NCH, K)
    zeros = jnp.zeros((N_PAD, FH), jnp.float32)
    p1 = _sc_spmm(x[:, :FH], x[:, FH:], src, dst, ev, zeros)
    h = _tc_mm_relu(p1, W1, b1)
    p2 = _sc_spmm(h[0], h[1], src, dst, ev, zeros)
    scores, g = _tc_final(p2, W2, b2, Wh, bh)
    return (scores, g)
